# Initial kernel scaffold; baseline (speedup 1.0000x reference)
import jax
import jax.numpy as jnp
from jax.experimental import pallas as pl


def kernel(x, cam_label, view_label, camera_embedding, view_embedding):
    raise NotImplementedError("write your pallas kernel here")


# speedup vs baseline: 1.8523x; 1.8523x over previous
import functools

import jax
import jax.numpy as jnp
from jax import lax
from jax.experimental import pallas as pl
from jax.experimental.pallas import tpu as pltpu
from jax.experimental.pallas import tpu_sc as plsc

B = 16384
C = 128
NC = 2
NS = 16
NW = NC * NS
BPW = B // NW
CH = 128
NCHUNK = BPW // CH
LANES = 16
CSEG = C // LANES


def _sie_body(x_hbm, cam_hbm, view_hbm, camtab_hbm, viewtab_hbm, out_hbm,
              cam_idx_v, view_idx_v, xbuf, cbuf, vbuf,
              sem_x, sem_c, sem_v):
    wid = lax.axis_index("s") * NC + lax.axis_index("c")
    base = wid * BPW

    pltpu.sync_copy(cam_hbm.at[pl.ds(base, BPW)], cam_idx_v)
    pltpu.sync_copy(view_hbm.at[pl.ds(base, BPW)], view_idx_v)

    def chunk(ci, carry):
        off = pl.multiple_of(ci * CH, CH)
        r0 = base + off
        cx = pltpu.async_copy(x_hbm.at[pl.ds(r0, CH)], xbuf, sem_x)
        cc = pltpu.async_copy(camtab_hbm.at[cam_idx_v.at[pl.ds(off, CH)]],
                              cbuf, sem_c)
        cv = pltpu.async_copy(viewtab_hbm.at[view_idx_v.at[pl.ds(off, CH)]],
                              vbuf, sem_v)
        cx.wait()
        cc.wait()
        cv.wait()

        def row(r, rcarry):
            for j in range(CSEG):
                sl = pl.ds(j * LANES, LANES)
                xbuf[r, sl] = xbuf[r, sl] + cbuf[r, sl] + vbuf[r, sl]
            return rcarry

        lax.fori_loop(0, CH, row, 0, unroll=2)
        pltpu.sync_copy(xbuf, out_hbm.at[pl.ds(r0, CH)])
        return carry

    lax.fori_loop(0, NCHUNK, chunk, 0)


@functools.partial(jax.jit, static_argnames=())
def _sie(x, cam_label, view_label, camera_embedding, view_embedding):
    run = pl.kernel(
        _sie_body,
        out_type=jax.ShapeDtypeStruct((B, C), jnp.float32),
        mesh=plsc.VectorSubcoreMesh(core_axis_name="c", subcore_axis_name="s"),
        scratch_types=[
            pltpu.VMEM((BPW,), jnp.int32),
            pltpu.VMEM((BPW,), jnp.int32),
            pltpu.VMEM((CH, C), jnp.float32),
            pltpu.VMEM((CH, C), jnp.float32),
            pltpu.VMEM((CH, C), jnp.float32),
            pltpu.SemaphoreType.DMA,
            pltpu.SemaphoreType.DMA,
            pltpu.SemaphoreType.DMA,
        ],
    )
    return run(x, cam_label, view_label, camera_embedding, view_embedding)


def kernel(x, cam_label, view_label, camera_embedding, view_embedding):
    return _sie(x, cam_label.astype(jnp.int32), view_label.astype(jnp.int32),
                camera_embedding, view_embedding)


# speedup vs baseline: 2.3697x; 1.2793x over previous
import functools

import jax
import jax.numpy as jnp
from jax import lax
from jax.experimental import pallas as pl
from jax.experimental.pallas import tpu as pltpu
from jax.experimental.pallas import tpu_sc as plsc

B = 16384
C = 128
NC = 2
NS = 16
NW = NC * NS
BPW = B // NW
CH = 128
NCHUNK = BPW // CH
LANES = 16
CSEG = C // LANES


def _sie_body(x_hbm, cam_hbm, view_hbm, camtab_hbm, viewtab_hbm, out_hbm,
              cam_idx_v, view_idx_v, xbuf,
              sem_x, sem_c, sem_v):
    wid = lax.axis_index("s") * NC + lax.axis_index("c")
    base = wid * BPW

    pltpu.sync_copy(cam_hbm.at[pl.ds(base, BPW)], cam_idx_v)
    pltpu.sync_copy(view_hbm.at[pl.ds(base, BPW)], view_idx_v)

    def chunk(ci, carry):
        off = pl.multiple_of(ci * CH, CH)
        r0 = base + off
        cx = pltpu.async_copy(x_hbm.at[pl.ds(r0, CH)], xbuf, sem_x)
        cx.wait()
        cc = pltpu.async_copy(camtab_hbm.at[cam_idx_v.at[pl.ds(off, CH)]],
                              xbuf, sem_c, add=True)
        cv = pltpu.async_copy(viewtab_hbm.at[view_idx_v.at[pl.ds(off, CH)]],
                              xbuf, sem_v, add=True)
        cc.wait()
        cv.wait()
        pltpu.sync_copy(xbuf, out_hbm.at[pl.ds(r0, CH)])
        return carry

    lax.fori_loop(0, NCHUNK, chunk, 0)


@functools.partial(jax.jit, static_argnames=())
def _sie(x, cam_label, view_label, camera_embedding, view_embedding):
    run = pl.kernel(
        _sie_body,
        out_type=jax.ShapeDtypeStruct((B, C), jnp.float32),
        mesh=plsc.VectorSubcoreMesh(core_axis_name="c", subcore_axis_name="s"),
        scratch_types=[
            pltpu.VMEM((BPW,), jnp.int32),
            pltpu.VMEM((BPW,), jnp.int32),
            pltpu.VMEM((CH, C), jnp.float32),
            pltpu.SemaphoreType.DMA,
            pltpu.SemaphoreType.DMA,
            pltpu.SemaphoreType.DMA,
        ],
    )
    return run(x, cam_label, view_label, camera_embedding, view_embedding)


def kernel(x, cam_label, view_label, camera_embedding, view_embedding):
    return _sie(x, cam_label.astype(jnp.int32), view_label.astype(jnp.int32),
                camera_embedding, view_embedding)


# speedup vs baseline: 2.3756x; 1.0025x over previous
import functools

import jax
import jax.numpy as jnp
from jax import lax
from jax.experimental import pallas as pl
from jax.experimental.pallas import tpu as pltpu
from jax.experimental.pallas import tpu_sc as plsc

B = 16384
C = 128
NC = 2
NS = 16
NW = NC * NS
BPW = B // NW
CH = 128
NCHUNK = BPW // CH
LANES = 16
CSEG = C // LANES


def _sie_body(x_hbm, cam_hbm, view_hbm, camtab_hbm, viewtab_hbm, out_hbm,
              cam_idx_v, view_idx_v, xbuf,
              sems_x, sems_c, sems_v, sems_o):
    wid = lax.axis_index("s") * NC + lax.axis_index("c")
    base = wid * BPW

    pltpu.sync_copy(cam_hbm.at[pl.ds(base, BPW)], cam_idx_v)
    pltpu.sync_copy(view_hbm.at[pl.ds(base, BPW)], view_idx_v)

    xc = []
    for i in range(NCHUNK):
        off = i * CH
        xc.append(pltpu.async_copy(x_hbm.at[pl.ds(base + off, CH)],
                                   xbuf.at[pl.ds(off, CH)], sems_x[i]))
    gathers = []
    for i in range(NCHUNK):
        off = i * CH
        xc[i].wait()
        cc = pltpu.async_copy(camtab_hbm.at[cam_idx_v.at[pl.ds(off, CH)]],
                              xbuf.at[pl.ds(off, CH)], sems_c[i], add=True)
        cv = pltpu.async_copy(viewtab_hbm.at[view_idx_v.at[pl.ds(off, CH)]],
                              xbuf.at[pl.ds(off, CH)], sems_v[i], add=True)
        gathers.append((cc, cv))
    wbs = []
    for i in range(NCHUNK):
        off = i * CH
        cc, cv = gathers[i]
        cc.wait()
        cv.wait()
        wbs.append(pltpu.async_copy(xbuf.at[pl.ds(off, CH)],
                                    out_hbm.at[pl.ds(base + off, CH)],
                                    sems_o[i]))
    for w in wbs:
        w.wait()


@functools.partial(jax.jit, static_argnames=())
def _sie(x, cam_label, view_label, camera_embedding, view_embedding):
    run = pl.kernel(
        _sie_body,
        out_type=jax.ShapeDtypeStruct((B, C), jnp.float32),
        mesh=plsc.VectorSubcoreMesh(core_axis_name="c", subcore_axis_name="s"),
        scratch_types=[
            pltpu.VMEM((BPW,), jnp.int32),
            pltpu.VMEM((BPW,), jnp.int32),
            pltpu.VMEM((BPW, C), jnp.float32),
            [pltpu.SemaphoreType.DMA] * NCHUNK,
            [pltpu.SemaphoreType.DMA] * NCHUNK,
            [pltpu.SemaphoreType.DMA] * NCHUNK,
            [pltpu.SemaphoreType.DMA] * NCHUNK,
        ],
    )
    return run(x, cam_label, view_label, camera_embedding, view_embedding)


def kernel(x, cam_label, view_label, camera_embedding, view_embedding):
    return _sie(x, cam_label.astype(jnp.int32), view_label.astype(jnp.int32),
                camera_embedding, view_embedding)


# speedup vs baseline: 2.4801x; 1.0440x over previous
import functools

import jax
import jax.numpy as jnp
from jax import lax
from jax.experimental import pallas as pl
from jax.experimental.pallas import tpu as pltpu
from jax.experimental.pallas import tpu_sc as plsc

B = 16384
C = 128
NC = 2
NS = 16
NW = NC * NS
BPW = B // NW


def _sie_body(x_hbm, cam_hbm, view_hbm, camtab_hbm, viewtab_hbm, out_hbm,
              cam_idx_v, view_idx_v, xbuf, sem_i, sem_x, sem_c, sem_v):
    wid = lax.axis_index("s") * NC + lax.axis_index("c")

    ci = pltpu.async_copy(cam_hbm.at[wid], cam_idx_v, sem_i)
    vi = pltpu.async_copy(view_hbm.at[wid], view_idx_v, sem_i)
    cx = pltpu.async_copy(x_hbm.at[wid], xbuf, sem_x)
    ci.wait()
    vi.wait()
    cx.wait()
    cc = pltpu.async_copy(camtab_hbm.at[cam_idx_v], xbuf, sem_c, add=True)
    cv = pltpu.async_copy(viewtab_hbm.at[view_idx_v], xbuf, sem_v, add=True)
    cc.wait()
    cv.wait()
    pltpu.sync_copy(xbuf, out_hbm.at[wid])


@functools.partial(jax.jit, static_argnames=())
def _sie(x, cam_label, view_label, camera_embedding, view_embedding):
    run = pl.kernel(
        _sie_body,
        out_type=jax.ShapeDtypeStruct((NW, BPW, C), jnp.float32),
        mesh=plsc.VectorSubcoreMesh(core_axis_name="c", subcore_axis_name="s"),
        scratch_types=[
            pltpu.VMEM((BPW,), jnp.int32),
            pltpu.VMEM((BPW,), jnp.int32),
            pltpu.VMEM((BPW, C), jnp.float32),
            pltpu.SemaphoreType.DMA,
            pltpu.SemaphoreType.DMA,
            pltpu.SemaphoreType.DMA,
            pltpu.SemaphoreType.DMA,
        ],
    )
    out = run(x.reshape(NW, BPW, C),
              cam_label.reshape(NW, BPW),
              view_label.reshape(NW, BPW),
              camera_embedding, view_embedding)
    return out.reshape(B, C)


def kernel(x, cam_label, view_label, camera_embedding, view_embedding):
    return _sie(x, cam_label.astype(jnp.int32), view_label.astype(jnp.int32),
                camera_embedding, view_embedding)


# speedup vs baseline: 3.9203x; 1.5807x over previous
import functools

import jax
import jax.numpy as jnp
from jax import lax
from jax.experimental import pallas as pl
from jax.experimental.pallas import tpu as pltpu
from jax.experimental.pallas import tpu_sc as plsc

B = 16384
C = 128
VIEW = 100
NC = 2
NS = 16
NW = NC * NS
BPW = B // NW


def _sie_body(x_hbm, cam_hbm, view_hbm, camtab_hbm, viewtab_hbm, out_hbm,
              cam_idx_v, view_idx_v, vtab_sh, xbuf,
              sem_i, sem_t, sem_x, sem_c, sem_v):
    wid = lax.axis_index("s") * NC + lax.axis_index("c")

    ci = pltpu.async_copy(cam_hbm.at[wid], cam_idx_v, sem_i)
    vi = pltpu.async_copy(view_hbm.at[wid], view_idx_v, sem_i)
    cx = pltpu.async_copy(x_hbm.at[wid], xbuf, sem_x)

    @pl.when(lax.axis_index("s") == 0)
    def _stage_view_table():
        pltpu.sync_copy(viewtab_hbm, vtab_sh)

    ci.wait()
    vi.wait()
    cx.wait()
    cc = pltpu.async_copy(camtab_hbm.at[cam_idx_v], xbuf, sem_c, add=True)
    plsc.subcore_barrier()
    cv = pltpu.async_copy(vtab_sh.at[view_idx_v], xbuf, sem_v, add=True)
    cc.wait()
    cv.wait()
    pltpu.sync_copy(xbuf, out_hbm.at[wid])


@functools.partial(jax.jit, static_argnames=())
def _sie(x, cam_label, view_label, camera_embedding, view_embedding):
    run = pl.kernel(
        _sie_body,
        out_type=jax.ShapeDtypeStruct((NW, BPW, C), jnp.float32),
        mesh=plsc.VectorSubcoreMesh(core_axis_name="c", subcore_axis_name="s"),
        scratch_types=[
            pltpu.VMEM((BPW,), jnp.int32),
            pltpu.VMEM((BPW,), jnp.int32),
            pltpu.VMEM_SHARED((VIEW, C), jnp.float32),
            pltpu.VMEM((BPW, C), jnp.float32),
            pltpu.SemaphoreType.DMA,
            pltpu.SemaphoreType.DMA,
            pltpu.SemaphoreType.DMA,
            pltpu.SemaphoreType.DMA,
            pltpu.SemaphoreType.DMA,
        ],
    )
    out = run(x.reshape(NW, BPW, C),
              cam_label.reshape(NW, BPW),
              view_label.reshape(NW, BPW),
              camera_embedding, view_embedding)
    return out.reshape(B, C)


def kernel(x, cam_label, view_label, camera_embedding, view_embedding):
    return _sie(x, cam_label.astype(jnp.int32), view_label.astype(jnp.int32),
                camera_embedding, view_embedding)
